# Initial kernel scaffold; baseline (speedup 1.0000x reference)
import jax
import jax.numpy as jnp
from jax.experimental import pallas as pl


def kernel(fields, edge_attr, W_msg, b_msg, degrees, edge_index):
    raise NotImplementedError("write your pallas kernel here")


# speedup vs baseline: 2.9340x; 2.9340x over previous
import jax
import jax.numpy as jnp
from jax.experimental import pallas as pl
from jax.experimental.pallas import tpu as pltpu

_NU = 0.01
_R = 8
_CC = 12500


def _final_body(u, v, gu0, gu1, gv0, gv1, gp0, gp1, lapu, lapv, out):
    out[0] = gu0[...] + gv1[...]
    out[1] = u[...] * gu0[...] + v[...] * gu1[...] + gp0[...] - _NU * lapu[...]
    out[2] = u[...] * gv0[...] + v[...] * gv1[...] + gp1[...] - _NU * lapv[...]


def kernel(fields, edge_attr, W_msg, b_msg, degrees, edge_index):
    N = fields.shape[0]
    src = edge_index[0]
    dst = edge_index[1]
    u = fields[:, 0]
    v = fields[:, 1]
    p = fields[:, 2]
    deg = degrees

    vals1 = jnp.concatenate(
        [fields[src], deg[src][:, None], edge_attr,
         jnp.ones((src.shape[0], 1), jnp.float32)], axis=1)
    acc1 = jax.ops.segment_sum(vals1, dst, num_segments=N)
    Su, Sv, Sp, Sd, K0, K1, C = [acc1[:, i] for i in range(7)]

    W = W_msg
    b = b_msg
    Qc = [W[1, j] * Sd + W[3, j] * C * deg + W[4, j] * K0 + W[5, j] * K1
          + b[j] * C for j in (0, 1)]

    def model_col(Sx, x, j):
        return (W[0, j] * Sx + W[2, j] * C * x + Qc[j]) / deg

    gu0 = model_col(Su, u, 0)
    gu1 = model_col(Su, u, 1)
    gv0 = model_col(Sv, v, 0)
    gv1 = model_col(Sv, v, 1)
    gp0 = model_col(Sp, p, 0)
    gp1 = model_col(Sp, p, 1)

    g4 = jnp.stack([gu0, gu1, gv0, gv1], axis=1)
    acc2 = jax.ops.segment_sum(g4[src], dst, num_segments=N)
    T0, T1, T2, T3 = [acc2[:, i] for i in range(4)]

    lap_u = (W[0, 0] * T0 + W[0, 1] * T1 + C * (W[2, 0] * gu0 + W[2, 1] * gu1)
             + Qc[0] + Qc[1]) / deg
    lap_v = (W[0, 0] * T2 + W[0, 1] * T3 + C * (W[2, 0] * gv0 + W[2, 1] * gv1)
             + Qc[0] + Qc[1]) / deg

    shp = (_R, _CC)
    args = [a.reshape(shp) for a in
            (u, v, gu0, gu1, gv0, gv1, gp0, gp1, lap_u, lap_v)]
    out3 = pl.pallas_call(
        _final_body,
        out_shape=jax.ShapeDtypeStruct((3, _R, _CC), jnp.float32),
    )(*args)
    return out3.reshape(3, N).T


# speedup vs baseline: 41.4538x; 14.1290x over previous
import functools

import jax
import jax.numpy as jnp
from jax import lax
from jax.experimental import pallas as pl
from jax.experimental.pallas import tpu as pltpu
from jax.experimental.pallas import tpu_sc as plsc

_NU = 0.01
_N = 100000
_E = 1600000
_NP = 102400
_NW = 32
_EPT = _E // _NW
_CHUNK = 2000
_NCH = _EPT // _CHUNK
_ZROWS = 3200
_TROWS = _NP // 16


def _make_edge_pass(with_ev: bool):
    outs = [jax.ShapeDtypeStruct((2, _NP, 8), jnp.float32)]
    scratch = [pltpu.VMEM_SHARED((_NP, 8), jnp.float32)]
    scratch += [
        pltpu.VMEM((_ZROWS, 8), jnp.float32),
        pltpu.VMEM((_CHUNK,), jnp.int32),
        pltpu.VMEM((_CHUNK,), jnp.int32),
        pltpu.VMEM((_CHUNK, 8), jnp.float32),
    ]
    if with_ev:
        scratch.append(pltpu.VMEM((_CHUNK, 8), jnp.float32))
    scratch.append(pltpu.SemaphoreType.DMA)
    mesh = plsc.VectorSubcoreMesh(core_axis_name="c", subcore_axis_name="s")

    def body(*refs):
        if with_ev:
            (tab, srcr, dstr, evr, zeros_h, out_a,
             acc_a, zbuf, src_v, dst_v, rows_v, ev_v, sem) = refs
        else:
            (tab, srcr, dstr, zeros_h, out_a,
             acc_a, zbuf, src_v, dst_v, rows_v, sem) = refs
            ev_v = evr = None
        cid = lax.axis_index("c")
        sid = lax.axis_index("s")
        wid = sid * 2 + cid

        pltpu.sync_copy(zeros_h, zbuf)
        zb = sid * _TROWS
        for j in range(_TROWS // _ZROWS):
            pltpu.sync_copy(zbuf, acc_a.at[pl.ds(zb + j * _ZROWS, _ZROWS)])
        plsc.subcore_barrier()

        ebase = wid * _EPT

        def step(t, carry):
            off = pl.multiple_of(ebase + t * _CHUNK, 8)
            pltpu.sync_copy(srcr.at[pl.ds(off, _CHUNK)], src_v)
            pltpu.sync_copy(dstr.at[pl.ds(off, _CHUNK)], dst_v)
            if with_ev:
                pltpu.sync_copy(evr.at[pl.ds(off, _CHUNK)], ev_v)
            pltpu.async_copy(tab.at[src_v], rows_v, sem).wait()
            pltpu.sync_copy(rows_v, acc_a.at[dst_v], add=True)
            if with_ev:
                pltpu.sync_copy(ev_v, acc_a.at[dst_v], add=True)
            return carry

        lax.fori_loop(0, _NCH, step, 0)
        plsc.subcore_barrier()

        for j in range(_TROWS // _ZROWS):
            r0 = zb + j * _ZROWS
            pltpu.sync_copy(acc_a.at[pl.ds(r0, _ZROWS)], zbuf)
            pltpu.sync_copy(zbuf, out_a.at[cid, pl.ds(r0, _ZROWS)])

    return functools.partial(
        pl.kernel, body, out_type=outs, mesh=mesh, scratch_types=scratch,
        compiler_params=pltpu.CompilerParams(use_tc_tiling_on_sc=False))()


_R = 8
_CC = 12500


def _final_body(wv, u, v, gu0, gu1, gv0, gv1, gp0, gp1,
                t0, t1, t2, t3, cc, dg, qc0, qc1, out):
    w00, w01, w20, w21 = wv[0], wv[1], wv[2], wv[3]
    qsum = qc0[...] + qc1[...]
    inv_d = 1.0 / dg[...]
    lap_u = (w00 * t0[...] + w01 * t1[...]
             + cc[...] * (w20 * gu0[...] + w21 * gu1[...]) + qsum) * inv_d
    lap_v = (w00 * t2[...] + w01 * t3[...]
             + cc[...] * (w20 * gv0[...] + w21 * gv1[...]) + qsum) * inv_d
    out[0] = gu0[...] + gv1[...]
    out[1] = u[...] * gu0[...] + v[...] * gu1[...] + gp0[...] - _NU * lap_u
    out[2] = u[...] * gv0[...] + v[...] * gv1[...] + gp1[...] - _NU * lap_v


def kernel(fields, edge_attr, W_msg, b_msg, degrees, edge_index):
    src = edge_index[0]
    dst = edge_index[1]
    u = fields[:, 0]
    v = fields[:, 1]
    p = fields[:, 2]
    deg = degrees
    W = W_msg
    b = b_msg

    tab1 = jnp.zeros((_NP, 8), jnp.float32).at[:_N, :4].set(
        jnp.concatenate([fields, deg[:, None]], axis=1))
    ev = jnp.concatenate(
        [jnp.zeros((_E, 4), jnp.float32), edge_attr,
         jnp.ones((_E, 1), jnp.float32), jnp.zeros((_E, 1), jnp.float32)],
        axis=1)
    zeros_h = jnp.zeros((_ZROWS, 8), jnp.float32)

    (pa,) = _make_edge_pass(True)(tab1, src, dst, ev, zeros_h)
    A = (pa[0] + pa[1])[:_N]
    Su, Sv, Sp, Sd = A[:, 0], A[:, 1], A[:, 2], A[:, 3]
    K0, K1, C = A[:, 4], A[:, 5], A[:, 6]

    Qc = [W[1, j] * Sd + W[3, j] * C * deg + W[4, j] * K0 + W[5, j] * K1
          + b[j] * C for j in (0, 1)]

    def model_col(Sx, x, j):
        return (W[0, j] * Sx + W[2, j] * C * x + Qc[j]) / deg

    gu0 = model_col(Su, u, 0)
    gu1 = model_col(Su, u, 1)
    gv0 = model_col(Sv, v, 0)
    gv1 = model_col(Sv, v, 1)
    gp0 = model_col(Sp, p, 0)
    gp1 = model_col(Sp, p, 1)

    tab2 = jnp.zeros((_NP, 8), jnp.float32).at[:_N, :4].set(
        jnp.stack([gu0, gu1, gv0, gv1], axis=1))
    (pt,) = _make_edge_pass(False)(tab2, src, dst, zeros_h)
    T = (pt[0] + pt[1])[:_N]

    wv = jnp.stack([W[0, 0], W[0, 1], W[2, 0], W[2, 1]])
    shp = (_R, _CC)
    args = [a.reshape(shp) for a in
            (u, v, gu0, gu1, gv0, gv1, gp0, gp1,
             T[:, 0], T[:, 1], T[:, 2], T[:, 3], C, deg, Qc[0], Qc[1])]
    out3 = pl.pallas_call(
        _final_body,
        in_specs=[pl.BlockSpec(memory_space=pltpu.SMEM)]
        + [pl.BlockSpec(shp, lambda: (0, 0))] * 16,
        out_specs=pl.BlockSpec((3, _R, _CC), lambda: (0, 0, 0)),
        out_shape=jax.ShapeDtypeStruct((3, _R, _CC), jnp.float32),
    )(wv, *args)
    return out3.reshape(3, _N).T
